# baseline (speedup 1.0000x reference)
import functools

import jax
import jax.numpy as jnp
from jax import lax
from jax.experimental import pallas as pl
from jax.experimental.pallas import tpu as pltpu
from jax.experimental.pallas import tpu_sc as plsc

N = 10000
C = 10000
E = 320000
D = 128

NC = 2
NS = 16
NW = NC * NS

K = 128
NCH = E // K
FULL = NCH // NW
REM = NCH % NW
ROWS_PER_TILE = 632
CPAD = NS * ROWS_PER_TILE


def _sc_body(x_hbm, inc_hbm, zeros_hbm, out_hbm,
             idx, rows, acc, semA, semB, semI0, semI1, semI2):
    cid = lax.axis_index("c")
    sid = lax.axis_index("s")
    wid = sid * NC + cid

    r0 = sid * ROWS_PER_TILE
    pltpu.sync_copy(zeros_hbm, acc.at[pl.ds(r0, ROWS_PER_TILE)])
    plsc.subcore_barrier()

    def chunk_off(c):
        return (c * NW + wid) * K

    semg = (semA, semB)
    semi = (semI0, semI1, semI2)

    def idx_load(c, s):
        pltpu.async_copy(inc_hbm.at[:, pl.ds(chunk_off(c), K)], idx.at[s],
                         semi[s])

    def idx_drain(s):
        pltpu.make_async_copy(inc_hbm.at[:, pl.ds(0, K)], idx.at[s],
                              semi[s]).wait()

    def gather(s, r):
        pltpu.async_copy(x_hbm.at[idx.at[s, 0]], rows.at[r], semg[r])

    def gather_drain(r):
        pltpu.make_async_copy(x_hbm.at[pl.ds(0, K)], rows.at[r],
                              semg[r]).wait()

    def scatter(r, s):
        pltpu.sync_copy(rows.at[r], acc.at[idx.at[s, 1]], add=True)

    idx_load(0, 0)
    idx_drain(0)
    gather(0, 0)
    idx_load(1, 1)
    idx_load(2, 2)

    def six(t, carry):
        c0 = 6 * t
        for i in range(6):
            c = c0 + i
            r = i % 2
            s = i % 3

            gather_drain(r)

            @pl.when(c + 1 < FULL)
            def _():
                idx_drain((i + 1) % 3)
                gather((i + 1) % 3, (i + 1) % 2)

            scatter(r, s)

            @pl.when(c + 3 < FULL)
            def _():
                idx_load(c + 3, s)
        return carry

    lax.fori_loop(0, FULL // 6, six, jnp.int32(0))

    @pl.when(wid < REM)
    def _():
        b = (FULL * NW + wid) * K
        pltpu.sync_copy(inc_hbm.at[:, pl.ds(b, K)], idx.at[0])
        pltpu.async_copy(x_hbm.at[idx.at[0, 0]], rows.at[0], semA).wait()
        pltpu.sync_copy(rows.at[0], acc.at[idx.at[0, 1]], add=True)
    plsc.subcore_barrier()

    pltpu.sync_copy(acc.at[pl.ds(r0, ROWS_PER_TILE)],
                    out_hbm.at[cid, pl.ds(r0, ROWS_PER_TILE)])


_sc_segment_sum = functools.partial(
    pl.kernel,
    out_type=jax.ShapeDtypeStruct((NC, CPAD, D), jnp.float32),
    mesh=plsc.VectorSubcoreMesh(
        core_axis_name="c", subcore_axis_name="s", num_cores=NC, num_subcores=NS
    ),
    scratch_types=[
        pltpu.VMEM((3, 2, K), jnp.int32),
        pltpu.VMEM((2, K, D), jnp.float32),
        pltpu.VMEM_SHARED((CPAD, D), jnp.float32),
        pltpu.SemaphoreType.DMA,
        pltpu.SemaphoreType.DMA,
        pltpu.SemaphoreType.DMA,
        pltpu.SemaphoreType.DMA,
        pltpu.SemaphoreType.DMA,
    ],
)(_sc_body)


BC = 2000


def _mlp_body(p_ref, w1_ref, b1_ref, w2_ref, b2_ref, o_ref):
    a = p_ref[0] + p_ref[1]
    h = jnp.dot(a, w1_ref[...], preferred_element_type=jnp.float32) + b1_ref[...]
    h = jnp.maximum(h, 0.0)
    o_ref[...] = (
        jnp.dot(h, w2_ref[...], preferred_element_type=jnp.float32) + b2_ref[...]
    )


_mlp = pl.pallas_call(
    _mlp_body,
    grid=(C // BC,),
    in_specs=[
        pl.BlockSpec((NC, BC, D), lambda i: (0, i, 0)),
        pl.BlockSpec((D, D), lambda i: (0, 0)),
        pl.BlockSpec((1, D), lambda i: (0, 0)),
        pl.BlockSpec((D, D), lambda i: (0, 0)),
        pl.BlockSpec((1, D), lambda i: (0, 0)),
    ],
    out_specs=pl.BlockSpec((BC, D), lambda i: (i, 0)),
    out_shape=jax.ShapeDtypeStruct((C, D), jnp.float32),
)


def kernel(x, incidence_index, W1, b1, W2, b2, eps):
    inc = incidence_index.astype(jnp.int32)
    zeros = jnp.zeros((ROWS_PER_TILE, D), dtype=jnp.float32)
    partials = _sc_segment_sum(x, inc, zeros)
    w1s = W1 * (1.0 + eps)
    return _mlp(partials, w1s, b1.reshape(1, D), W2, b2.reshape(1, D))


# speedup vs baseline: 1.0464x; 1.0464x over previous
import functools

import jax
import jax.numpy as jnp
from jax import lax
from jax.experimental import pallas as pl
from jax.experimental.pallas import tpu as pltpu
from jax.experimental.pallas import tpu_sc as plsc

N = 10000
C = 10000
E = 320000
D = 128

NC = 2
NS = 16
NW = NC * NS

K = 128
NCH = E // K
FULL = NCH // NW
REM = NCH % NW
ROWS_PER_TILE = 632
CPAD = NS * ROWS_PER_TILE


def _sc_body(x_hbm, inc_hbm, zeros_hbm, out_hbm,
             idx, rows, acc, semA, semB):
    cid = lax.axis_index("c")
    sid = lax.axis_index("s")
    wid = sid * NC + cid

    def chunk_off(c):
        return (c * NW + wid) * K

    semg = (semA, semB)

    def idx_load(c, slot):
        pltpu.sync_copy(inc_hbm.at[:, pl.ds(chunk_off(c), K)], idx.at[slot])

    def gather(slot):
        pltpu.async_copy(x_hbm.at[idx.at[slot, 0]], rows.at[slot], semg[slot])

    def gather_drain(slot):
        pltpu.make_async_copy(x_hbm.at[pl.ds(0, K)], rows.at[slot],
                              semg[slot]).wait()

    def scatter(slot):
        pltpu.sync_copy(rows.at[slot], acc.at[idx.at[slot, 1]], add=True)

    NP = FULL // 2
    r0 = sid * ROWS_PER_TILE
    z = pltpu.async_copy(zeros_hbm, acc.at[pl.ds(r0, ROWS_PER_TILE)], semB)
    idx_load(0, 0)
    gather(0)
    idx_load(1, 1)
    z.wait()
    plsc.subcore_barrier()

    def pair(t, carry):
        gather(1)
        gather_drain(0)
        scatter(0)

        @pl.when(t < NP - 1)
        def _():
            idx_load(2 * t + 2, 0)
            gather(0)

        gather_drain(1)
        scatter(1)

        @pl.when(t < NP - 1)
        def _():
            idx_load(2 * t + 3, 1)

        return carry

    lax.fori_loop(0, NP, pair, jnp.int32(0))

    @pl.when(wid < REM)
    def _():
        b = (FULL * NW + wid) * K
        pltpu.sync_copy(inc_hbm.at[:, pl.ds(b, K)], idx.at[0])
        pltpu.async_copy(x_hbm.at[idx.at[0, 0]], rows.at[0], semA).wait()
        pltpu.sync_copy(rows.at[0], acc.at[idx.at[0, 1]], add=True)
    plsc.subcore_barrier()

    pltpu.sync_copy(acc.at[pl.ds(r0, ROWS_PER_TILE)],
                    out_hbm.at[cid, pl.ds(r0, ROWS_PER_TILE)])


_sc_segment_sum = functools.partial(
    pl.kernel,
    out_type=jax.ShapeDtypeStruct((NC, CPAD, D), jnp.float32),
    mesh=plsc.VectorSubcoreMesh(
        core_axis_name="c", subcore_axis_name="s", num_cores=NC, num_subcores=NS
    ),
    scratch_types=[
        pltpu.VMEM((2, 2, K), jnp.int32),
        pltpu.VMEM((2, K, D), jnp.float32),
        pltpu.VMEM_SHARED((CPAD, D), jnp.float32),
        pltpu.SemaphoreType.DMA,
        pltpu.SemaphoreType.DMA,
    ],
)(_sc_body)


BC = 2000


def _mlp_body(p_ref, w1_ref, b1_ref, w2_ref, b2_ref, o_ref):
    a = p_ref[0] + p_ref[1]
    h = jnp.dot(a, w1_ref[...], preferred_element_type=jnp.float32) + b1_ref[...]
    h = jnp.maximum(h, 0.0)
    o_ref[...] = (
        jnp.dot(h, w2_ref[...], preferred_element_type=jnp.float32) + b2_ref[...]
    )


_mlp = pl.pallas_call(
    _mlp_body,
    grid=(C // BC,),
    in_specs=[
        pl.BlockSpec((NC, BC, D), lambda i: (0, i, 0)),
        pl.BlockSpec((D, D), lambda i: (0, 0)),
        pl.BlockSpec((1, D), lambda i: (0, 0)),
        pl.BlockSpec((D, D), lambda i: (0, 0)),
        pl.BlockSpec((1, D), lambda i: (0, 0)),
    ],
    out_specs=pl.BlockSpec((BC, D), lambda i: (i, 0)),
    out_shape=jax.ShapeDtypeStruct((C, D), jnp.float32),
)


def kernel(x, incidence_index, W1, b1, W2, b2, eps):
    inc = incidence_index.astype(jnp.int32)
    zeros = jnp.zeros((ROWS_PER_TILE, D), dtype=jnp.float32)
    partials = _sc_segment_sum(x, inc, zeros)
    w1s = W1 * (1.0 + eps)
    return _mlp(partials, w1s, b1.reshape(1, D), W2, b2.reshape(1, D))
